# baseline (speedup 1.0000x reference)
import functools

import jax
import jax.numpy as jnp
from jax import lax
from jax.experimental import pallas as pl
from jax.experimental.pallas import tpu as pltpu
from jax.experimental.pallas import tpu_sc as plsc

C = 16
S3 = 8
NW = 32
CIN = 256


def _expansion_patterns():
    lanes = lax.iota(jnp.int32, 16)
    l8 = lanes // 8
    j = lanes % 8
    offc = [(j >> 2) & 1, (j >> 1) & 1, j & 1]
    rowv = [jnp.full((16,), r, jnp.int32) for r in range(C)]
    return l8, offc, rowv


def _expand(src_d, src_i, dst_d, dst_i, ncols, pats):
    l8, offc, rowv = pats

    def gather_group(gbase, kk):
        idx = gbase + (kk * 2 + l8)
        xs = [plsc.load_gather(src_d, [rowv[ch], idx]) for ch in range(C)]
        ys = [plsc.load_gather(src_i, [rowv[r], idx]) * 2 + offc[r]
              for r in range(3)]
        return xs + ys

    def store_group(ob, vals):
        for ch in range(C):
            dst_d[ch, pl.ds(ob, 16)] = vals[ch]
        for r in range(3):
            dst_i[r, pl.ds(ob, 16)] = vals[C + r]

    def g_body(g, carry):
        gbase = g * 16
        prev = gather_group(gbase, 0)
        for kk in range(1, S3):
            cur = gather_group(gbase, kk)
            store_group(g * 128 + (kk - 1) * 16, prev)
            prev = cur
        store_group(g * 128 + (S3 - 1) * 16, prev)
        return carry

    lax.fori_loop(0, ncols // 16, g_body, 0)


def _make_sc_upsample(n):
    naligned = (n // 128) * 128
    tail = n - naligned
    assert naligned % CIN == 0
    nfull = naligned // CIN
    mesh = plsc.VectorSubcoreMesh(core_axis_name="c", subcore_axis_name="s")

    scratch = [
        pltpu.VMEM((C, CIN), jnp.float32),
        pltpu.VMEM((3, CIN), jnp.int32),
        pltpu.VMEM((C, CIN * S3), jnp.float32),
        pltpu.VMEM((C, CIN * S3), jnp.float32),
        pltpu.VMEM((3, CIN * S3), jnp.int32),
        pltpu.VMEM((3, CIN * S3), jnp.int32),
        pltpu.SemaphoreType.DMA((6,)),
    ]
    if tail:
        scratch += [
            pltpu.VMEM((C, tail), jnp.float32),
            pltpu.VMEM((3, tail), jnp.int32),
            pltpu.VMEM((C, tail * S3), jnp.float32),
            pltpu.VMEM((3, tail * S3), jnp.int32),
        ]

    @functools.partial(
        pl.kernel,
        mesh=mesh,
        out_type=[
            jax.ShapeDtypeStruct((C, n * S3), jnp.float32),
            jax.ShapeDtypeStruct((3, n * S3), jnp.int32),
        ],
        scratch_types=scratch,
        compiler_params=pltpu.CompilerParams(needs_layout_passes=False),
    )
    def sc_upsample(data_hbm, ijk_hbm, dtail_hbm, itail_hbm,
                    out_data_hbm, out_ijk_hbm,
                    dbuf, ibuf, odbuf0, odbuf1, oibuf0, oibuf1, sem,
                    *tailbufs):
        wid = lax.axis_index("s") * 2 + lax.axis_index("c")
        pats = _expansion_patterns()
        odbufs = (odbuf0, odbuf1)
        oibufs = (oibuf0, oibuf1)

        def out_slices(c0):
            return (out_data_hbm.at[:, pl.ds(c0 * S3, CIN * S3)],
                    out_ijk_hbm.at[:, pl.ds(c0 * S3, CIN * S3)])

        def do_main(i, h):
            k = wid + i * NW

            @pl.when(k < nfull)
            def _():
                c0 = k * CIN
                cin_d = pltpu.async_copy(
                    data_hbm.at[:, pl.ds(c0, CIN)], dbuf, sem.at[0])
                cin_i = pltpu.async_copy(
                    ijk_hbm.at[:, pl.ds(c0, CIN)], ibuf, sem.at[1])
                cin_d.wait()
                cin_i.wait()
                od_hbm, oi_hbm = out_slices(c0)

                @pl.when(i >= 2)
                def _():
                    pltpu.make_async_copy(
                        odbufs[h], od_hbm, sem.at[2 + h]).wait()
                    pltpu.make_async_copy(
                        oibufs[h], oi_hbm, sem.at[4 + h]).wait()

                _expand(dbuf, ibuf, odbufs[h], oibufs[h], CIN, pats)
                pltpu.async_copy(odbufs[h], od_hbm, sem.at[2 + h])
                pltpu.async_copy(oibufs[h], oi_hbm, sem.at[4 + h])

        def do_tail(i):
            if not tail:
                return
            tdbuf, tibuf, todbuf, toibuf = tailbufs
            k = wid + i * NW

            @pl.when(k == nfull)
            def _():
                pltpu.sync_copy(dtail_hbm, tdbuf)
                pltpu.sync_copy(itail_hbm, tibuf)
                _expand(tdbuf, tibuf, todbuf, toibuf, tail, pats)
                pltpu.sync_copy(
                    todbuf,
                    out_data_hbm.at[:, pl.ds(naligned * S3, tail * S3)])
                pltpu.sync_copy(
                    toibuf,
                    out_ijk_hbm.at[:, pl.ds(naligned * S3, tail * S3)])

        nchunks = nfull + (1 if tail else 0)
        nit = (nchunks + NW - 1) // NW

        def pair_body(i2, carry):
            for h in (0, 1):
                i = 2 * i2 + h
                do_main(i, h)
                do_tail(i)
            return carry

        lax.fori_loop(0, (nit + 1) // 2, pair_body, 0)

        od_hbm, oi_hbm = out_slices(0)
        for h in (0, 1):
            pltpu.make_async_copy(odbufs[h], od_hbm, sem.at[2 + h]).wait()
            pltpu.make_async_copy(oibufs[h], oi_hbm, sem.at[4 + h]).wait()

    return sc_upsample


def kernel(coarse_data, coarse_ijk):
    n = coarse_data.shape[0]
    naligned = (n // 128) * 128
    dt = coarse_data.T
    it = coarse_ijk.T
    dtail = lax.slice(dt, (0, naligned), (C, n))
    itail = lax.slice(it, (0, naligned), (3, n))
    fn = _make_sc_upsample(n)
    fine_data_t, fine_ijk_t = fn(dt, it, dtail, itail)
    return fine_data_t.T, fine_ijk_t.T


# speedup vs baseline: 1.2848x; 1.2848x over previous
import functools

import jax
import jax.numpy as jnp
from jax import lax
from jax.experimental import pallas as pl
from jax.experimental.pallas import tpu as pltpu
from jax.experimental.pallas import tpu_sc as plsc

C = 16
S3 = 8
NW = 32
CIN = 256


def _expansion_patterns():
    lanes = lax.iota(jnp.int32, 16)
    l8 = lanes // 8
    j = lanes % 8
    offc = [(j >> 2) & 1, (j >> 1) & 1, j & 1]
    rowv = [jnp.full((16,), r, jnp.int32) for r in range(C)]
    return l8, offc, rowv


def _expand(src_d, src_i, dst_d, dst_i, ncols, pats):
    l8, offc, rowv = pats

    def gather_group(gbase, kk):
        idx = gbase + (kk * 2 + l8)
        xs = [plsc.load_gather(src_d, [rowv[ch], idx]) for ch in range(C)]
        ys = [plsc.load_gather(src_i, [rowv[r], idx]) * 2 + offc[r]
              for r in range(3)]
        return xs + ys

    def store_group(ob, vals):
        for ch in range(C):
            dst_d[ch, pl.ds(ob, 16)] = vals[ch]
        for r in range(3):
            dst_i[r, pl.ds(ob, 16)] = vals[C + r]

    def g_body(g, carry):
        gbase = g * 16
        prev = gather_group(gbase, 0)
        for kk in range(1, S3):
            cur = gather_group(gbase, kk)
            store_group(g * 128 + (kk - 1) * 16, prev)
            prev = cur
        store_group(g * 128 + (S3 - 1) * 16, prev)
        return carry

    lax.fori_loop(0, ncols // 16, g_body, 0)


def _make_sc_upsample(n):
    naligned = (n // 128) * 128
    tail = n - naligned
    assert naligned % CIN == 0
    nfull = naligned // CIN
    mesh = plsc.VectorSubcoreMesh(core_axis_name="c", subcore_axis_name="s")

    scratch = [
        pltpu.VMEM((C, CIN), jnp.float32),
        pltpu.VMEM((C, CIN), jnp.float32),
        pltpu.VMEM((3, CIN), jnp.int32),
        pltpu.VMEM((3, CIN), jnp.int32),
        pltpu.VMEM((C, CIN * S3), jnp.float32),
        pltpu.VMEM((C, CIN * S3), jnp.float32),
        pltpu.VMEM((3, CIN * S3), jnp.int32),
        pltpu.VMEM((3, CIN * S3), jnp.int32),
        pltpu.SemaphoreType.DMA((8,)),
    ]
    if tail:
        scratch += [
            pltpu.VMEM((C, tail), jnp.float32),
            pltpu.VMEM((3, tail), jnp.int32),
            pltpu.VMEM((C, tail * S3), jnp.float32),
            pltpu.VMEM((3, tail * S3), jnp.int32),
        ]

    @functools.partial(
        pl.kernel,
        mesh=mesh,
        out_type=[
            jax.ShapeDtypeStruct((C, n * S3), jnp.float32),
            jax.ShapeDtypeStruct((3, n * S3), jnp.int32),
        ],
        scratch_types=scratch,
        compiler_params=pltpu.CompilerParams(needs_layout_passes=False),
    )
    def sc_upsample(data_hbm, ijk_hbm, dtail_hbm, itail_hbm,
                    out_data_hbm, out_ijk_hbm,
                    dbuf0, dbuf1, ibuf0, ibuf1,
                    odbuf0, odbuf1, oibuf0, oibuf1, sem,
                    *tailbufs):
        wid = lax.axis_index("s") * 2 + lax.axis_index("c")
        pats = _expansion_patterns()
        dbufs = (dbuf0, dbuf1)
        ibufs = (ibuf0, ibuf1)
        odbufs = (odbuf0, odbuf1)
        oibufs = (oibuf0, oibuf1)

        def out_slices(c0):
            return (out_data_hbm.at[:, pl.ds(c0 * S3, CIN * S3)],
                    out_ijk_hbm.at[:, pl.ds(c0 * S3, CIN * S3)])

        def issue_in(k, h):
            c0 = k * CIN
            pltpu.async_copy(
                data_hbm.at[:, pl.ds(c0, CIN)], dbufs[h], sem.at[0 + h])
            pltpu.async_copy(
                ijk_hbm.at[:, pl.ds(c0, CIN)], ibufs[h], sem.at[2 + h])

        def do_main(i, h):
            k = wid + i * NW

            @pl.when(k + NW < nfull)
            def _():
                issue_in(k + NW, 1 - h)

            @pl.when(k < nfull)
            def _():
                c0 = k * CIN
                od_hbm, oi_hbm = out_slices(c0)
                pltpu.make_async_copy(
                    data_hbm.at[:, pl.ds(c0, CIN)], dbufs[h],
                    sem.at[0 + h]).wait()
                pltpu.make_async_copy(
                    ijk_hbm.at[:, pl.ds(c0, CIN)], ibufs[h],
                    sem.at[2 + h]).wait()

                @pl.when(i >= 2)
                def _():
                    pltpu.make_async_copy(
                        odbufs[h], od_hbm, sem.at[4 + h]).wait()
                    pltpu.make_async_copy(
                        oibufs[h], oi_hbm, sem.at[6 + h]).wait()

                _expand(dbufs[h], ibufs[h], odbufs[h], oibufs[h], CIN,
                        pats)
                pltpu.async_copy(odbufs[h], od_hbm, sem.at[4 + h])
                pltpu.async_copy(oibufs[h], oi_hbm, sem.at[6 + h])

        def do_tail(i):
            if not tail:
                return
            tdbuf, tibuf, todbuf, toibuf = tailbufs
            k = wid + i * NW

            @pl.when(k == nfull)
            def _():
                pltpu.sync_copy(dtail_hbm, tdbuf)
                pltpu.sync_copy(itail_hbm, tibuf)
                _expand(tdbuf, tibuf, todbuf, toibuf, tail, pats)
                pltpu.sync_copy(
                    todbuf,
                    out_data_hbm.at[:, pl.ds(naligned * S3, tail * S3)])
                pltpu.sync_copy(
                    toibuf,
                    out_ijk_hbm.at[:, pl.ds(naligned * S3, tail * S3)])

        nchunks = nfull + (1 if tail else 0)
        nit = (nchunks + NW - 1) // NW

        @pl.when(wid < nfull)
        def _():
            issue_in(wid, 0)

        def pair_body(i2, carry):
            for h in (0, 1):
                i = 2 * i2 + h
                do_main(i, h)
                do_tail(i)
            return carry

        lax.fori_loop(0, (nit + 1) // 2, pair_body, 0)

        od_hbm, oi_hbm = out_slices(0)
        for h in (0, 1):
            pltpu.make_async_copy(odbufs[h], od_hbm, sem.at[4 + h]).wait()
            pltpu.make_async_copy(oibufs[h], oi_hbm, sem.at[6 + h]).wait()

    return sc_upsample


def kernel(coarse_data, coarse_ijk):
    n = coarse_data.shape[0]
    naligned = (n // 128) * 128
    dt = coarse_data.T
    it = coarse_ijk.T
    dtail = lax.slice(dt, (0, naligned), (C, n))
    itail = lax.slice(it, (0, naligned), (3, n))
    fn = _make_sc_upsample(n)
    fine_data_t, fine_ijk_t = fn(dt, it, dtail, itail)
    return fine_data_t.T, fine_ijk_t.T
